# baseline (speedup 1.0000x reference)
import jax
import jax.numpy as jnp
from jax import lax
from jax.experimental import pallas as pl
from jax.experimental.pallas import tpu as pltpu
from jax.experimental.pallas import tpu_sc as plsc

NSEG = 1024
LANES = 16
NC = 2
NS = 16
NW = NC * NS


def _make_sc_kernel(N, D, C, interpret=False):
    SPW = NSEG // NW
    DF = 3 * D
    NCH = DF // LANES
    DCH = D // LANES
    assert N % LANES == 0 and N % C == 0 and C == LANES

    def body(h0, h1, h2, idxh, out, idx_v, b0, b1, b2, acc, cnt, sem0, sem1):
        sems = (sem0, sem1)
        bufs = (b0, b1, b2)
        cid = lax.axis_index("c")
        sid = lax.axis_index("s")
        w = sid * NC + cid
        seg_lo = w * SPW
        seg_hi = seg_lo + SPW

        pltpu.sync_copy(idxh, idx_v.at[pl.ds(0, N)])

        zero16 = jnp.zeros((LANES,), jnp.float32)

        def zero_body(i, _):
            for ch in range(NCH):
                acc[i, pl.ds(ch * LANES, LANES)] = zero16
            cnt[i, pl.ds(0, LANES)] = zero16
            return 0

        lax.fori_loop(0, SPW, zero_body, 0)

        def lower_bound(x):
            def bisect(_, carry):
                lo_b, hi_b = carry
                active = lo_b < hi_b
                mid = (lo_b + hi_b) // 2
                v = idx_v[pl.ds(mid, LANES)][0]
                lt = jnp.logical_and(active, v < x)
                ge = jnp.logical_and(active, jnp.logical_not(v < x))
                lo_b = jnp.where(lt, mid + 1, lo_b)
                hi_b = jnp.where(ge, mid, hi_b)
                return lo_b, hi_b

            lo_b, _ = lax.fori_loop(
                0, 17, bisect, (jnp.int32(0), jnp.int32(N)))
            return lo_b

        lo = lower_bound(seg_lo)
        hi = lower_bound(seg_hi)

        k0 = lo // C
        k1 = (hi + (C - 1)) // C

        def start(k, half):
            base = k * C
            for h, b in zip((h0, h1, h2), bufs):
                pltpu.async_copy(h.at[pl.ds(base, C)], b.at[half], sems[half])

        def wait(half):
            for h, b in zip((h0, h1, h2), bufs):
                pltpu.make_async_copy(h.at[pl.ds(0, C)], b.at[half], sems[half]).wait()

        @pl.when(k0 < k1)
        def _():
            start(k0, 0)

        @pl.when(k0 + 1 < k1)
        def _():
            start(k0 + 1, 1)

        one16 = jnp.ones((LANES,), jnp.float32)

        def process(k, half):
            rbase = k * C

            @plsc.parallel_loop(jnp.maximum(lo, rbase),
                                jnp.minimum(hi, rbase + C), unroll=4)
            def _(r):
                slot = idx_v[pl.ds(r, LANES)][0] - seg_lo
                rr = r - rbase
                for j, b in enumerate(bufs):
                    for ch in range(DCH):
                        v = b[half, rr, pl.ds(ch * LANES, LANES)]
                        plsc.addupdate(
                            acc.at[slot, pl.ds(j * D + ch * LANES, LANES)], v)
                plsc.addupdate(cnt.at[slot, pl.ds(0, LANES)], one16)

        def pair_body(q, _):
            for half in (0, 1):
                k = k0 + 2 * q + half

                @pl.when(k < k1)
                def _():
                    wait(half)
                    process(k, half)

                    @pl.when(k + 2 < k1)
                    def _():
                        start(k + 2, half)
            return 0

        lax.fori_loop(0, (k1 - k0 + 1) // 2, pair_body, 0)

        def fin_body(i, _):
            cvec = cnt[i, pl.ds(0, LANES)]
            inv = 1.0 / jnp.maximum(cvec, 1.0)
            for ch in range(NCH):
                acc[i, pl.ds(ch * LANES, LANES)] = (
                    acc[i, pl.ds(ch * LANES, LANES)] * inv)
            return 0

        lax.fori_loop(0, SPW, fin_body, 0)
        pltpu.sync_copy(acc, out.at[pl.ds(seg_lo, SPW)])

    mesh = plsc.VectorSubcoreMesh(
        core_axis_name="c", subcore_axis_name="s",
        num_cores=NC, num_subcores=NS)
    return pl.kernel(
        body,
        out_type=jax.ShapeDtypeStruct((NSEG, DF), jnp.float32),
        mesh=mesh,
        scratch_types=[
            pltpu.VMEM((N + LANES,), jnp.int32),
            pltpu.VMEM((2, C, D), jnp.float32),
            pltpu.VMEM((2, C, D), jnp.float32),
            pltpu.VMEM((2, C, D), jnp.float32),
            pltpu.VMEM((NSEG // NW, 3 * D), jnp.float32),
            pltpu.VMEM((NSEG // NW, LANES), jnp.float32),
            pltpu.SemaphoreType.DMA,
            pltpu.SemaphoreType.DMA,
        ],
        interpret=interpret,
    )


def kernel(h0, h1, h2, index):
    N, D = h0.shape
    fn = _make_sc_kernel(N, D, C=16)
    return fn(h0, h1, h2, index.astype(jnp.int32))


# speedup vs baseline: 1.9181x; 1.9181x over previous
import jax
import jax.numpy as jnp
from jax import lax
from jax.experimental import pallas as pl
from jax.experimental.pallas import tpu as pltpu
from jax.experimental.pallas import tpu_sc as plsc

NSEG = 1024
LANES = 16
NC = 2
NS = 16
NW = NC * NS


def _make_sc_kernel(N, D, C, interpret=False):
    SPW = NSEG // NW
    DF = 3 * D
    NCH = DF // LANES
    DCH = D // LANES
    assert N % LANES == 0 and N % C == 0 and C == LANES

    def body(h0, h1, h2, idxh, out, idx_v, b0, b1, b2, acc, cnt, sem0, sem1):
        sems = (sem0, sem1)
        bufs = (b0, b1, b2)
        cid = lax.axis_index("c")
        sid = lax.axis_index("s")
        w = sid * NC + cid
        seg_lo = w * SPW
        seg_hi = seg_lo + SPW

        pltpu.sync_copy(idxh, idx_v.at[pl.ds(0, N)])

        zero16 = jnp.zeros((LANES,), jnp.float32)

        def zero_body(i, _):
            for ch in range(NCH):
                acc[i, pl.ds(ch * LANES, LANES)] = zero16
            cnt[i, pl.ds(0, LANES)] = zero16
            return 0

        lax.fori_loop(0, SPW, zero_body, 0)

        def lower_bound(x):
            def bisect(_, carry):
                lo_b, hi_b = carry
                active = lo_b < hi_b
                mid = (lo_b + hi_b) // 2
                v = idx_v[pl.ds(mid, LANES)][0]
                lt = jnp.logical_and(active, v < x)
                ge = jnp.logical_and(active, jnp.logical_not(v < x))
                lo_b = jnp.where(lt, mid + 1, lo_b)
                hi_b = jnp.where(ge, mid, hi_b)
                return lo_b, hi_b

            lo_b, _ = lax.fori_loop(
                0, 17, bisect, (jnp.int32(0), jnp.int32(N)))
            return lo_b

        lo = lower_bound(seg_lo)
        hi = lower_bound(seg_hi)

        k0 = lo // C
        k1 = (hi + (C - 1)) // C

        def start(k, half):
            base = k * C
            for h, b in zip((h0, h1, h2), bufs):
                pltpu.async_copy(h.at[pl.ds(base, C)], b.at[half], sems[half])

        def wait(half):
            for h, b in zip((h0, h1, h2), bufs):
                pltpu.make_async_copy(h.at[pl.ds(0, C)], b.at[half], sems[half]).wait()

        @pl.when(k0 < k1)
        def _():
            start(k0, 0)

        @pl.when(k0 + 1 < k1)
        def _():
            start(k0 + 1, 1)

        one16 = jnp.ones((LANES,), jnp.float32)

        def process(k, half):
            rbase = k * C

            @plsc.parallel_loop(jnp.maximum(lo, rbase),
                                jnp.minimum(hi, rbase + C), unroll=2)
            def _(r):
                slot = idx_v[pl.ds(r, LANES)][0] - seg_lo
                rr = r - rbase
                G = 8
                for j, b in enumerate(bufs):
                    for g in range(DCH // G):
                        vals = [b[half, rr, pl.ds((g * G + t) * LANES, LANES)]
                                for t in range(G)]
                        for t in range(G):
                            plsc.addupdate(
                                acc.at[slot,
                                       pl.ds(j * D + (g * G + t) * LANES,
                                             LANES)],
                                vals[t])
                plsc.addupdate(cnt.at[slot, pl.ds(0, LANES)], one16)

        def pair_body(q, _):
            for half in (0, 1):
                k = k0 + 2 * q + half

                @pl.when(k < k1)
                def _():
                    wait(half)
                    process(k, half)

                    @pl.when(k + 2 < k1)
                    def _():
                        start(k + 2, half)
            return 0

        lax.fori_loop(0, (k1 - k0 + 1) // 2, pair_body, 0)

        def fin_body(i, _):
            cvec = cnt[i, pl.ds(0, LANES)]
            inv = 1.0 / jnp.maximum(cvec, 1.0)
            for ch in range(NCH):
                acc[i, pl.ds(ch * LANES, LANES)] = (
                    acc[i, pl.ds(ch * LANES, LANES)] * inv)
            return 0

        lax.fori_loop(0, SPW, fin_body, 0)
        pltpu.sync_copy(acc, out.at[pl.ds(seg_lo, SPW)])

    mesh = plsc.VectorSubcoreMesh(
        core_axis_name="c", subcore_axis_name="s",
        num_cores=NC, num_subcores=NS)
    return pl.kernel(
        body,
        out_type=jax.ShapeDtypeStruct((NSEG, DF), jnp.float32),
        mesh=mesh,
        scratch_types=[
            pltpu.VMEM((N + LANES,), jnp.int32),
            pltpu.VMEM((2, C, D), jnp.float32),
            pltpu.VMEM((2, C, D), jnp.float32),
            pltpu.VMEM((2, C, D), jnp.float32),
            pltpu.VMEM((NSEG // NW, 3 * D), jnp.float32),
            pltpu.VMEM((NSEG // NW, LANES), jnp.float32),
            pltpu.SemaphoreType.DMA,
            pltpu.SemaphoreType.DMA,
        ],
        interpret=interpret,
    )


def kernel(h0, h1, h2, index):
    N, D = h0.shape
    fn = _make_sc_kernel(N, D, C=16)
    return fn(h0, h1, h2, index.astype(jnp.int32))


# speedup vs baseline: 1.9211x; 1.0016x over previous
import jax
import jax.numpy as jnp
from jax import lax
from jax.experimental import pallas as pl
from jax.experimental.pallas import tpu as pltpu
from jax.experimental.pallas import tpu_sc as plsc

NSEG = 1024
LANES = 16
NC = 2
NS = 16
NW = NC * NS


def _make_sc_kernel(N, D, C, interpret=False):
    SPW = NSEG // NW
    DF = 3 * D
    NCH = DF // LANES
    DCH = D // LANES
    assert N % LANES == 0 and N % C == 0 and C == LANES

    def body(h0, h1, h2, idxh, out, idx_v, b0, b1, b2, acc, cnt, sem0, sem1):
        sems = (sem0, sem1)
        bufs = (b0, b1, b2)
        cid = lax.axis_index("c")
        sid = lax.axis_index("s")
        w = sid * NC + cid
        seg_lo = w * SPW
        seg_hi = seg_lo + SPW

        pltpu.sync_copy(idxh, idx_v.at[pl.ds(0, N)])

        zero16 = jnp.zeros((LANES,), jnp.float32)

        def zero_body(i, _):
            for ch in range(NCH):
                acc[i, pl.ds(ch * LANES, LANES)] = zero16
            cnt[i, pl.ds(0, LANES)] = zero16
            return 0

        lax.fori_loop(0, SPW, zero_body, 0)

        def lower_bound(x):
            def bisect(_, carry):
                lo_b, hi_b = carry
                active = lo_b < hi_b
                mid = (lo_b + hi_b) // 2
                v = idx_v[pl.ds(mid, LANES)][0]
                lt = jnp.logical_and(active, v < x)
                ge = jnp.logical_and(active, jnp.logical_not(v < x))
                lo_b = jnp.where(lt, mid + 1, lo_b)
                hi_b = jnp.where(ge, mid, hi_b)
                return lo_b, hi_b

            lo_b, _ = lax.fori_loop(
                0, 17, bisect, (jnp.int32(0), jnp.int32(N)))
            return lo_b

        lo = lower_bound(seg_lo)
        hi = lower_bound(seg_hi)

        k0 = lo // C
        k1 = (hi + (C - 1)) // C

        def start(k, half):
            base = k * C
            for h, b in zip((h0, h1, h2), bufs):
                pltpu.async_copy(h.at[pl.ds(base, C)], b.at[half], sems[half])

        def wait(half):
            for h, b in zip((h0, h1, h2), bufs):
                pltpu.make_async_copy(h.at[pl.ds(0, C)], b.at[half], sems[half]).wait()

        @pl.when(k0 < k1)
        def _():
            start(k0, 0)

        @pl.when(k0 + 1 < k1)
        def _():
            start(k0 + 1, 1)

        one16 = jnp.ones((LANES,), jnp.float32)

        def process(k, half):
            rbase = k * C

            @plsc.parallel_loop(jnp.maximum(lo, rbase),
                                jnp.minimum(hi, rbase + C), unroll=2)
            def _(r):
                slot = idx_v[pl.ds(r, LANES)][0] - seg_lo
                rr = r - rbase
                G = 16
                for j, b in enumerate(bufs):
                    for g in range(DCH // G):
                        vals = [b[half, rr, pl.ds((g * G + t) * LANES, LANES)]
                                for t in range(G)]
                        for t in range(G):
                            plsc.addupdate(
                                acc.at[slot,
                                       pl.ds(j * D + (g * G + t) * LANES,
                                             LANES)],
                                vals[t])
                plsc.addupdate(cnt.at[slot, pl.ds(0, LANES)], one16)

        def pair_body(q, _):
            for half in (0, 1):
                k = k0 + 2 * q + half

                @pl.when(k < k1)
                def _():
                    wait(half)
                    process(k, half)

                    @pl.when(k + 2 < k1)
                    def _():
                        start(k + 2, half)
            return 0

        lax.fori_loop(0, (k1 - k0 + 1) // 2, pair_body, 0)

        def fin_body(i, _):
            cvec = cnt[i, pl.ds(0, LANES)]
            inv = 1.0 / jnp.maximum(cvec, 1.0)
            for ch in range(NCH):
                acc[i, pl.ds(ch * LANES, LANES)] = (
                    acc[i, pl.ds(ch * LANES, LANES)] * inv)
            return 0

        lax.fori_loop(0, SPW, fin_body, 0)
        pltpu.sync_copy(acc, out.at[pl.ds(seg_lo, SPW)])

    mesh = plsc.VectorSubcoreMesh(
        core_axis_name="c", subcore_axis_name="s",
        num_cores=NC, num_subcores=NS)
    return pl.kernel(
        body,
        out_type=jax.ShapeDtypeStruct((NSEG, DF), jnp.float32),
        mesh=mesh,
        scratch_types=[
            pltpu.VMEM((N + LANES,), jnp.int32),
            pltpu.VMEM((2, C, D), jnp.float32),
            pltpu.VMEM((2, C, D), jnp.float32),
            pltpu.VMEM((2, C, D), jnp.float32),
            pltpu.VMEM((NSEG // NW, 3 * D), jnp.float32),
            pltpu.VMEM((NSEG // NW, LANES), jnp.float32),
            pltpu.SemaphoreType.DMA,
            pltpu.SemaphoreType.DMA,
        ],
        interpret=interpret,
    )


def kernel(h0, h1, h2, index):
    N, D = h0.shape
    fn = _make_sc_kernel(N, D, C=16)
    return fn(h0, h1, h2, index.astype(jnp.int32))


# speedup vs baseline: 2.1059x; 1.0962x over previous
import jax
import jax.numpy as jnp
from jax import lax
from jax.experimental import pallas as pl
from jax.experimental.pallas import tpu as pltpu
from jax.experimental.pallas import tpu_sc as plsc

NSEG = 1024
LANES = 16
NC = 2
NS = 16
NW = NC * NS


def _make_sc_kernel(N, D, C, interpret=False):
    SPW = NSEG // NW
    DF = 3 * D
    NCH = DF // LANES
    DCH = D // LANES
    assert N % LANES == 0 and N % C == 0 and C == LANES

    def body(h0, h1, h2, idxh, out, idx_v, b0, b1, b2, acc, bnd, sem0, sem1):
        sems = (sem0, sem1)
        bufs = (b0, b1, b2)
        cid = lax.axis_index("c")
        sid = lax.axis_index("s")
        w = sid * NC + cid
        seg_lo = w * SPW
        seg_hi = seg_lo + SPW

        pltpu.sync_copy(idxh, idx_v.at[pl.ds(0, N)])

        zero16 = jnp.zeros((LANES,), jnp.float32)

        def zero_body(i, _):
            for ch in range(NCH):
                acc[i, pl.ds(ch * LANES, LANES)] = zero16
            return 0

        lax.fori_loop(0, SPW, zero_body, 0)

        def lower_bound(x):
            def bisect(_, carry):
                lo_b, hi_b = carry
                active = lo_b < hi_b
                mid = (lo_b + hi_b) // 2
                v = idx_v[pl.ds(mid, LANES)][0]
                lt = jnp.logical_and(active, v < x)
                ge = jnp.logical_and(active, jnp.logical_not(v < x))
                lo_b = jnp.where(lt, mid + 1, lo_b)
                hi_b = jnp.where(ge, mid, hi_b)
                return lo_b, hi_b

            lo_b, _ = lax.fori_loop(
                0, 17, bisect, (jnp.int32(0), jnp.int32(N)))
            return lo_b

        def bnd_body(s, _):
            bnd[s] = lower_bound(seg_lo + s)
            return 0

        lax.fori_loop(0, SPW + 1, bnd_body, 0)
        lo = bnd[0]
        hi = bnd[SPW]

        k0 = lo // C
        k1 = (hi + (C - 1)) // C

        def start(k, half):
            base = k * C
            for h, b in zip((h0, h1, h2), bufs):
                pltpu.async_copy(h.at[pl.ds(base, C)], b.at[half], sems[half])

        def wait(half):
            for h, b in zip((h0, h1, h2), bufs):
                pltpu.make_async_copy(h.at[pl.ds(0, C)], b.at[half], sems[half]).wait()

        @pl.when(k0 < k1)
        def _():
            start(k0, 0)

        @pl.when(k0 + 1 < k1)
        def _():
            start(k0 + 1, 1)

        zeros48 = tuple(zero16 for _ in range(NCH))

        def process(k, half):
            rbase = k * C
            r0 = jnp.maximum(lo, rbase)
            r1 = jnp.minimum(hi, rbase + C)
            nonempty = r0 < r1

            s_a = idx_v[pl.ds(r0, LANES)][0] - seg_lo
            s_b = idx_v[pl.ds(jnp.maximum(r1 - 1, 0), LANES)][0] - seg_lo + 1
            s_first = jnp.where(nonempty, s_a, 0)
            s_end = jnp.where(nonempty, s_b, 0)

            def srun(s, _):
                ra = jnp.maximum(bnd[s], r0)
                rb = jnp.minimum(bnd[s + 1], r1)

                @plsc.parallel_loop(ra, rb, carry=zeros48)
                def run_sum(r, carry):
                    rr = r - rbase
                    vals = []
                    for j, b in enumerate(bufs):
                        for ch in range(DCH):
                            vals.append(b[half, rr, pl.ds(ch * LANES, LANES)])
                    return tuple(carry[i] + vals[i] for i in range(NCH))

                for i in range(NCH):
                    plsc.addupdate(acc.at[s, pl.ds(i * LANES, LANES)],
                                   run_sum[i])
                return 0

            lax.fori_loop(s_first, s_end, srun, 0)

        def pair_body(q, _):
            for half in (0, 1):
                k = k0 + 2 * q + half

                @pl.when(k < k1)
                def _():
                    wait(half)
                    process(k, half)

                    @pl.when(k + 2 < k1)
                    def _():
                        start(k + 2, half)
            return 0

        lax.fori_loop(0, (k1 - k0 + 1) // 2, pair_body, 0)

        def fin_body(s, _):
            cf = (bnd[s + 1] - bnd[s]).astype(jnp.float32)
            cvec = lax.broadcast_in_dim(cf, (LANES,), ())
            inv = 1.0 / jnp.maximum(cvec, 1.0)
            for ch in range(NCH):
                acc[s, pl.ds(ch * LANES, LANES)] = (
                    acc[s, pl.ds(ch * LANES, LANES)] * inv)
            return 0

        lax.fori_loop(0, SPW, fin_body, 0)
        pltpu.sync_copy(acc, out.at[pl.ds(seg_lo, SPW)])

    mesh = plsc.VectorSubcoreMesh(
        core_axis_name="c", subcore_axis_name="s",
        num_cores=NC, num_subcores=NS)
    return pl.kernel(
        body,
        out_type=jax.ShapeDtypeStruct((NSEG, DF), jnp.float32),
        mesh=mesh,
        scratch_types=[
            pltpu.VMEM((N + LANES,), jnp.int32),
            pltpu.VMEM((2, C, D), jnp.float32),
            pltpu.VMEM((2, C, D), jnp.float32),
            pltpu.VMEM((2, C, D), jnp.float32),
            pltpu.VMEM((NSEG // NW, 3 * D), jnp.float32),
            pltpu.SMEM((48,), jnp.int32),
            pltpu.SemaphoreType.DMA,
            pltpu.SemaphoreType.DMA,
        ],
        interpret=interpret,
    )


def kernel(h0, h1, h2, index):
    N, D = h0.shape
    fn = _make_sc_kernel(N, D, C=16)
    return fn(h0, h1, h2, index.astype(jnp.int32))


# speedup vs baseline: 2.4295x; 1.1537x over previous
import jax
import jax.numpy as jnp
from jax import lax
from jax.experimental import pallas as pl
from jax.experimental.pallas import tpu as pltpu
from jax.experimental.pallas import tpu_sc as plsc

NSEG = 1024
LANES = 16
NC = 2
NS = 16
NW = NC * NS


def _make_sc_kernel(N, D, C, interpret=False):
    SPW = NSEG // NW
    DF = 3 * D
    NCH = DF // LANES
    DCH = D // LANES
    assert N % LANES == 0 and C % 8 == 0 and (N - C) % 8 == 0

    def body(h0, h1, h2, idxh, out, idx_v, b0, b1, b2, acc, bnd, sem0, sem1):
        sems = (sem0, sem1)
        bufs = (b0, b1, b2)
        cid = lax.axis_index("c")
        sid = lax.axis_index("s")
        w = sid * NC + cid
        seg_lo = w * SPW
        seg_hi = seg_lo + SPW

        pltpu.sync_copy(idxh, idx_v.at[pl.ds(0, N)])

        zero16 = jnp.zeros((LANES,), jnp.float32)

        def zero_body(i, _):
            for ch in range(NCH):
                acc[i, pl.ds(ch * LANES, LANES)] = zero16
            return 0

        lax.fori_loop(0, SPW, zero_body, 0)

        def lower_bound(x):
            def bisect(_, carry):
                lo_b, hi_b = carry
                active = lo_b < hi_b
                mid = (lo_b + hi_b) // 2
                v = idx_v[pl.ds(mid, LANES)][0]
                lt = jnp.logical_and(active, v < x)
                ge = jnp.logical_and(active, jnp.logical_not(v < x))
                lo_b = jnp.where(lt, mid + 1, lo_b)
                hi_b = jnp.where(ge, mid, hi_b)
                return lo_b, hi_b

            lo_b, _ = lax.fori_loop(
                0, 17, bisect, (jnp.int32(0), jnp.int32(N)))
            return lo_b

        def bnd_body(s, _):
            bnd[s] = lower_bound(seg_lo + s)
            return 0

        lax.fori_loop(0, SPW + 1, bnd_body, 0)
        lo = bnd[0]
        hi = bnd[SPW]

        k0 = lo // C
        k1 = (hi + (C - 1)) // C

        def start(k, half):
            base = jnp.minimum(k * C, N - C)
            for h, b in zip((h0, h1, h2), bufs):
                pltpu.async_copy(h.at[pl.ds(base, C)], b.at[half], sems[half])

        def wait(half):
            for h, b in zip((h0, h1, h2), bufs):
                pltpu.make_async_copy(h.at[pl.ds(0, C)], b.at[half], sems[half]).wait()

        @pl.when(k0 < k1)
        def _():
            start(k0, 0)

        @pl.when(k0 + 1 < k1)
        def _():
            start(k0 + 1, 1)

        zeros48 = tuple(zero16 for _ in range(NCH))

        def process(k, half):
            rbase = jnp.minimum(k * C, N - C)
            r0 = jnp.maximum(lo, k * C)
            r1 = jnp.minimum(hi, k * C + C)
            nonempty = r0 < r1

            s_a = idx_v[pl.ds(r0, LANES)][0] - seg_lo
            s_b = idx_v[pl.ds(jnp.maximum(r1 - 1, 0), LANES)][0] - seg_lo + 1
            s_first = jnp.where(nonempty, s_a, 0)
            s_end = jnp.where(nonempty, s_b, 0)

            def srun(s, _):
                ra = jnp.maximum(bnd[s], r0)
                rb = jnp.minimum(bnd[s + 1], r1)

                @plsc.parallel_loop(ra, rb, carry=zeros48)
                def run_sum(r, carry):
                    rr = r - rbase
                    vals = []
                    for j, b in enumerate(bufs):
                        for ch in range(DCH):
                            vals.append(b[half, rr, pl.ds(ch * LANES, LANES)])
                    return tuple(carry[i] + vals[i] for i in range(NCH))

                for i in range(NCH):
                    plsc.addupdate(acc.at[s, pl.ds(i * LANES, LANES)],
                                   run_sum[i])
                return 0

            lax.fori_loop(s_first, s_end, srun, 0)

        def pair_body(q, _):
            for half in (0, 1):
                k = k0 + 2 * q + half

                @pl.when(k < k1)
                def _():
                    wait(half)
                    process(k, half)

                    @pl.when(k + 2 < k1)
                    def _():
                        start(k + 2, half)
            return 0

        lax.fori_loop(0, (k1 - k0 + 1) // 2, pair_body, 0)

        def fin_body(s, _):
            cf = (bnd[s + 1] - bnd[s]).astype(jnp.float32)
            cvec = lax.broadcast_in_dim(cf, (LANES,), ())
            inv = 1.0 / jnp.maximum(cvec, 1.0)
            for ch in range(NCH):
                acc[s, pl.ds(ch * LANES, LANES)] = (
                    acc[s, pl.ds(ch * LANES, LANES)] * inv)
            return 0

        lax.fori_loop(0, SPW, fin_body, 0)
        pltpu.sync_copy(acc, out.at[pl.ds(seg_lo, SPW)])

    mesh = plsc.VectorSubcoreMesh(
        core_axis_name="c", subcore_axis_name="s",
        num_cores=NC, num_subcores=NS)
    return pl.kernel(
        body,
        out_type=jax.ShapeDtypeStruct((NSEG, DF), jnp.float32),
        mesh=mesh,
        scratch_types=[
            pltpu.VMEM((N + LANES,), jnp.int32),
            pltpu.VMEM((2, C, D), jnp.float32),
            pltpu.VMEM((2, C, D), jnp.float32),
            pltpu.VMEM((2, C, D), jnp.float32),
            pltpu.VMEM((NSEG // NW, 3 * D), jnp.float32),
            pltpu.SMEM((48,), jnp.int32),
            pltpu.SemaphoreType.DMA,
            pltpu.SemaphoreType.DMA,
        ],
        interpret=interpret,
    )


def kernel(h0, h1, h2, index):
    N, D = h0.shape
    fn = _make_sc_kernel(N, D, C=32)
    return fn(h0, h1, h2, index.astype(jnp.int32))
